# baseline (speedup 1.0000x reference)
import functools

import jax
import jax.numpy as jnp
from jax import lax
from jax.experimental import pallas as pl
from jax.experimental.pallas import tpu as pltpu
from jax.experimental.pallas import tpu_sc as plsc


def _sc_gather(tok_table, idx_flat):
    info = plsc.get_sparse_core_info()
    nw = info.num_cores * info.num_subcores
    n = idx_flat.shape[0]
    d = tok_table.shape[1]
    b_per_w = n // nw
    mesh = plsc.VectorSubcoreMesh(core_axis_name="c", subcore_axis_name="s")

    @functools.partial(
        pl.kernel,
        mesh=mesh,
        out_type=jax.ShapeDtypeStruct((n, d), jnp.float32),
        scratch_types=[
            pltpu.VMEM((b_per_w,), jnp.int32),
            pltpu.VMEM((b_per_w, d), jnp.float32),
            pltpu.SemaphoreType.DMA,
        ],
        compiler_params=pltpu.CompilerParams(use_tc_tiling_on_sc=False),
    )
    def gather_k(table_hbm, idx_hbm, out_hbm, idx_v, rows_v, sem):
        nc = info.num_cores
        wid = lax.axis_index("s") * nc + lax.axis_index("c")
        base = wid * b_per_w
        pltpu.sync_copy(idx_hbm.at[pl.ds(base, b_per_w)], idx_v)
        pltpu.async_copy(table_hbm.at[idx_v], rows_v, sem).wait()
        pltpu.sync_copy(rows_v, out_hbm.at[pl.ds(base, b_per_w)])

    return gather_k(tok_table, idx_flat)


_VT = 2048


def _head_body(x_ref, pos_ref, w_ref, b_ref, o_ref):
    x = x_ref[...] + pos_ref[...]
    o_ref[...] = (
        jnp.dot(x, w_ref[...], preferred_element_type=jnp.float32) + b_ref[...]
    )


def _head(x_tok, pos_full, W, b2):
    m, d = x_tok.shape
    v = W.shape[1]
    nv = pl.cdiv(v, _VT)
    return pl.pallas_call(
        _head_body,
        grid=(nv,),
        in_specs=[
            pl.BlockSpec((m, d), lambda j: (0, 0)),
            pl.BlockSpec((m, d), lambda j: (0, 0)),
            pl.BlockSpec((d, _VT), lambda j: (0, j)),
            pl.BlockSpec((1, _VT), lambda j: (0, j)),
        ],
        out_specs=pl.BlockSpec((m, _VT), lambda j: (0, j)),
        out_shape=jax.ShapeDtypeStruct((m, v), jnp.float32),
        compiler_params=pltpu.CompilerParams(
            dimension_semantics=("arbitrary",),
        ),
    )(x_tok, pos_full, W, b2)


def kernel(idx, tok_table, pos_table, W, b):
    bb, t = idx.shape
    d = tok_table.shape[1]
    idx_flat = idx.reshape(-1).astype(jnp.int32)
    x_tok = _sc_gather(tok_table, idx_flat)
    pos_full = jnp.broadcast_to(pos_table[None], (bb, t, d)).reshape(bb * t, d)
    out = _head(x_tok, pos_full, W, b.reshape(1, -1))
    return out.reshape(bb, t, -1)


# speedup vs baseline: 1.0628x; 1.0628x over previous
import functools

import jax
import jax.numpy as jnp
from jax import lax
from jax.experimental import pallas as pl
from jax.experimental.pallas import tpu as pltpu
from jax.experimental.pallas import tpu_sc as plsc


def _sc_gather(tok_table, idx_flat):
    info = plsc.get_sparse_core_info()
    nw = info.num_cores * info.num_subcores
    n = idx_flat.shape[0]
    d = tok_table.shape[1]
    b_per_w = n // nw
    mesh = plsc.VectorSubcoreMesh(core_axis_name="c", subcore_axis_name="s")

    @functools.partial(
        pl.kernel,
        mesh=mesh,
        out_type=jax.ShapeDtypeStruct((n, d), jnp.float32),
        scratch_types=[
            pltpu.VMEM((b_per_w,), jnp.int32),
            pltpu.VMEM((b_per_w, d), jnp.float32),
            pltpu.SemaphoreType.DMA,
        ],
    )
    def gather_k(table_hbm, idx_hbm, out_hbm, idx_v, rows_v, sem):
        nc = info.num_cores
        wid = lax.axis_index("s") * nc + lax.axis_index("c")
        base = wid * b_per_w
        pltpu.sync_copy(idx_hbm.at[pl.ds(base, b_per_w)], idx_v)
        vecs = [idx_v[pl.ds(c * 16, 16)] for c in range(b_per_w // 16)]
        copies = [
            pltpu.async_copy(
                table_hbm.at[pl.ds(vecs[i // 16][i % 16], 1)],
                rows_v.at[pl.ds(i, 1)],
                sem,
            )
            for i in range(b_per_w)
        ]
        for c in copies:
            c.wait()
        pltpu.sync_copy(rows_v, out_hbm.at[pl.ds(base, b_per_w)])

    return gather_k(tok_table, idx_flat)


_VT = 2048


def _head_body(x_ref, pos_ref, w_ref, b_ref, o_ref):
    x = x_ref[...] + pos_ref[...]
    o_ref[...] = (
        jnp.dot(x, w_ref[...], preferred_element_type=jnp.float32) + b_ref[...]
    )


def _head(x_tok, pos_full, W, b2):
    m, d = x_tok.shape
    v = W.shape[1]
    nv = pl.cdiv(v, _VT)
    return pl.pallas_call(
        _head_body,
        grid=(nv,),
        in_specs=[
            pl.BlockSpec((m, d), lambda j: (0, 0)),
            pl.BlockSpec((m, d), lambda j: (0, 0)),
            pl.BlockSpec((d, _VT), lambda j: (0, j)),
            pl.BlockSpec((1, _VT), lambda j: (0, j)),
        ],
        out_specs=pl.BlockSpec((m, _VT), lambda j: (0, j)),
        out_shape=jax.ShapeDtypeStruct((m, v), jnp.float32),
        compiler_params=pltpu.CompilerParams(
            dimension_semantics=("arbitrary",),
        ),
    )(x_tok, pos_full, W, b2)


def kernel(idx, tok_table, pos_table, W, b):
    bb, t = idx.shape
    d = tok_table.shape[1]
    idx_flat = idx.reshape(-1).astype(jnp.int32)
    x_tok = _sc_gather(tok_table, idx_flat)
    pos_full = jnp.broadcast_to(pos_table[None], (bb, t, d)).reshape(bb * t, d)
    out = _head(x_tok, pos_full, W, b.reshape(1, -1))
    return out.reshape(bb, t, -1)
